# baseline (speedup 1.0000x reference)
import functools

import jax
import jax.numpy as jnp
from jax.experimental import pallas as pl
from jax.experimental.pallas import tpu as pltpu

_T = 8


def _sigmoid(x):
    return 0.5 * jnp.tanh(0.5 * x) + 0.5


def _lstm_body(x_ref, p_ref, wx_ref, wh_ref, b_ref, h_out, c_out,
               xp_ref, xs_ref, h_ref, c_ref, *, T, B, U):
    it = pl.program_id(0)
    wr = jax.lax.rem(it, 2)
    rd = jax.lax.rem(it + 1, 2)

    @pl.when(it <= 1)
    def _init():
        h_ref[...] = jnp.zeros_like(h_ref)
        c_ref[...] = jnp.zeros_like(c_ref)

    for t in range(T):
        h_b = h_ref[...]
        gates = xp_ref[rd, pl.ds(t * B, B), :].astype(jnp.float32) + jnp.dot(
            h_b, wh_ref[...], preferred_element_type=jnp.float32)
        i = _sigmoid(gates[:, :U])
        f = _sigmoid(gates[:, U:2 * U])
        g = jnp.tanh(gates[:, 2 * U:3 * U])
        o = _sigmoid(gates[:, 3 * U:])
        c_new = f * c_ref[...] + i * g
        c_ref[...] = c_new
        h_new = o * jnp.tanh(c_new)
        h_ref[...] = h_new.astype(jnp.bfloat16)
        if t == T - 1:
            h_out[...] = h_new
            c_out[...] = c_new

    xs_b = x_ref[...].astype(jnp.bfloat16).reshape(B * T, x_ref.shape[2])
    xs_ref[...] = jnp.dot(p_ref[...], xs_b,
                          preferred_element_type=jnp.float32
                          ).astype(jnp.bfloat16)
    xpv = (
        jnp.dot(xs_ref[...], wx_ref[...], preferred_element_type=jnp.float32)
        + b_ref[...]
    ).astype(jnp.bfloat16)
    for j in range(T):
        sl = pl.ds(j * B, B)
        xp_ref[wr, sl, :] = xpv[j * B:(j + 1) * B, :]


@jax.jit
def kernel(x, Wx, Wh, b):
    B, S, D = x.shape
    U = Wh.shape[0]
    G = 4 * U
    T = _T
    NC = S // T

    wx = Wx.astype(jnp.bfloat16)
    wh = Wh.astype(jnp.bfloat16)
    b2 = b.astype(jnp.float32).reshape(1, G)
    rows = jnp.arange(T * B)
    src = (rows % B) * T + rows // B
    perm = (src[:, None] == jnp.arange(B * T)[None, :]).astype(jnp.bfloat16)

    body = functools.partial(_lstm_body, T=T, B=B, U=U)
    h, c = pl.pallas_call(
        body,
        out_shape=[
            jax.ShapeDtypeStruct((B, U), jnp.float32),
            jax.ShapeDtypeStruct((B, U), jnp.float32),
        ],
        grid=(NC + 1,),
        in_specs=[
            pl.BlockSpec((B, T, D),
                         lambda it: (0, jnp.minimum(it, NC - 1), 0)),
            pl.BlockSpec((T * B, T * B), lambda it: (0, 0)),
            pl.BlockSpec((D, G), lambda it: (0, 0)),
            pl.BlockSpec((U, G), lambda it: (0, 0)),
            pl.BlockSpec((1, G), lambda it: (0, 0)),
        ],
        out_specs=[
            pl.BlockSpec((B, U), lambda it: (0, 0)),
            pl.BlockSpec((B, U), lambda it: (0, 0)),
        ],
        scratch_shapes=[
            pltpu.VMEM((2, T * B, G), jnp.bfloat16),
            pltpu.VMEM((T * B, D), jnp.bfloat16),
            pltpu.VMEM((B, U), jnp.bfloat16),
            pltpu.VMEM((B, U), jnp.float32),
        ],
        compiler_params=pltpu.CompilerParams(
            dimension_semantics=("arbitrary",),
            vmem_limit_bytes=56 * 1024 * 1024,
        ),
        name="lstm_fused",
    )(x, perm, wx, wh, b2)
    return h, c


# speedup vs baseline: 1.0047x; 1.0047x over previous
import functools

import jax
import jax.numpy as jnp
from jax.experimental import pallas as pl
from jax.experimental.pallas import tpu as pltpu

_T = 8
_GW = 256


def _sigmoid(x):
    return 0.5 * jnp.tanh(0.5 * x) + 0.5


def _lstm_body(x_ref, p_ref, wx_ref, wh_ref, b_ref, h_out, c_out,
               xp_ref, h_ref, c_ref, *, T, B, U):
    it = pl.program_id(0)
    nt = pl.num_programs(0)
    ng = U // _GW

    @pl.when(it == 0)
    def _init():
        h_ref[...] = jnp.zeros_like(h_ref)
        c_ref[...] = jnp.zeros_like(c_ref)

    xs_b = x_ref[...].astype(jnp.bfloat16).reshape(B * T, x_ref.shape[2])
    xs_t = jnp.dot(p_ref[...], xs_b,
                   preferred_element_type=jnp.float32).astype(jnp.bfloat16)
    xp_ref[...] = (
        jnp.dot(xs_t, wx_ref[...], preferred_element_type=jnp.float32)
        + b_ref[...]
    )

    h_val = h_ref[...]
    c_val = c_ref[...]
    for t in range(T):
        gates = xp_ref[pl.ds(t * B, B), :] + jnp.dot(
            h_val, wh_ref[...], preferred_element_type=jnp.float32)
        h_chunks = []
        c_chunks = []
        for n in range(ng):
            blk = gates[:, n * 4 * _GW:(n + 1) * 4 * _GW]
            i = _sigmoid(blk[:, 0 * _GW:1 * _GW])
            f = _sigmoid(blk[:, 1 * _GW:2 * _GW])
            g = jnp.tanh(blk[:, 2 * _GW:3 * _GW])
            o = _sigmoid(blk[:, 3 * _GW:4 * _GW])
            c_n = f * c_val[:, n * _GW:(n + 1) * _GW] + i * g
            c_chunks.append(c_n)
            h_chunks.append(o * jnp.tanh(c_n))
        c_val = jnp.concatenate(c_chunks, axis=1)
        h_f32 = jnp.concatenate(h_chunks, axis=1)
        h_val = h_f32.astype(jnp.bfloat16)

    h_ref[...] = h_val
    c_ref[...] = c_val

    @pl.when(it == nt - 1)
    def _write():
        h_out[...] = h_f32
        c_out[...] = c_val


@jax.jit
def kernel(x, Wx, Wh, b):
    B, S, D = x.shape
    U = Wh.shape[0]
    G = 4 * U
    T = _T

    cols = jnp.arange(G)
    n = cols // (4 * _GW)
    q = (cols % (4 * _GW)) // _GW
    off = cols % _GW
    src = q * U + n * _GW + off
    wx = Wx.astype(jnp.bfloat16)[:, src]
    wh = Wh.astype(jnp.bfloat16)[:, src]
    b2 = b.astype(jnp.float32)[src].reshape(1, G)
    rows = jnp.arange(T * B)
    rsrc = (rows % B) * T + rows // B
    perm = (rsrc[:, None] == jnp.arange(B * T)[None, :]).astype(jnp.bfloat16)

    body = functools.partial(_lstm_body, T=T, B=B, U=U)
    h, c = pl.pallas_call(
        body,
        out_shape=[
            jax.ShapeDtypeStruct((B, U), jnp.float32),
            jax.ShapeDtypeStruct((B, U), jnp.float32),
        ],
        grid=(S // T,),
        in_specs=[
            pl.BlockSpec((B, T, D), lambda it: (0, it, 0)),
            pl.BlockSpec((T * B, T * B), lambda it: (0, 0)),
            pl.BlockSpec((D, G), lambda it: (0, 0)),
            pl.BlockSpec((U, G), lambda it: (0, 0)),
            pl.BlockSpec((1, G), lambda it: (0, 0)),
        ],
        out_specs=[
            pl.BlockSpec((B, U), lambda it: (0, 0)),
            pl.BlockSpec((B, U), lambda it: (0, 0)),
        ],
        scratch_shapes=[
            pltpu.VMEM((T * B, G), jnp.float32),
            pltpu.VMEM((B, U), jnp.bfloat16),
            pltpu.VMEM((B, U), jnp.float32),
        ],
        compiler_params=pltpu.CompilerParams(
            dimension_semantics=("arbitrary",),
            vmem_limit_bytes=56 * 1024 * 1024,
        ),
        name="lstm_fused",
    )(x, perm, wx, wh, b2)
    return h, c
